# Initial kernel scaffold; baseline (speedup 1.0000x reference)
import jax
import jax.numpy as jnp
from jax.experimental import pallas as pl


def kernel(hidden_states, gate_up_proj, down_proj, router_w):
    raise NotImplementedError("write your pallas kernel here")


# speedup vs baseline: 4.3292x; 4.3292x over previous
import functools

import jax
import jax.numpy as jnp
from jax import lax
from jax.experimental import pallas as pl
from jax.experimental.pallas import tpu as pltpu

_E = 64
_TOPK = 2
_TILE = 128


def _router_body(x_ref, rw_ref, idx_ref, w_ref):
    x = x_ref[...]
    rw = rw_ref[...]
    logits = lax.dot_general(
        x, rw, (((1,), (1,)), ((), ())), preferred_element_type=jnp.float32
    )
    e = logits.shape[1]
    iota = lax.broadcasted_iota(jnp.int32, logits.shape, 1)
    m1 = jnp.max(logits, axis=1, keepdims=True)
    a1 = jnp.min(jnp.where(logits == m1, iota, e), axis=1, keepdims=True)
    masked = jnp.where(iota == a1, -jnp.inf, logits)
    m2 = jnp.max(masked, axis=1, keepdims=True)
    a2 = jnp.min(jnp.where(masked == m2, iota, e), axis=1, keepdims=True)
    w1 = jax.nn.sigmoid(m1 - m2)
    idx_ref[...] = jnp.concatenate([a1, a2], axis=1)
    w_ref[...] = jnp.concatenate([w1, 1.0 - w1], axis=1)


def _route(flat, router_w):
    n, d = flat.shape
    bt = 1024
    grid = n // bt
    idxs, ws = pl.pallas_call(
        _router_body,
        grid=(grid,),
        in_specs=[
            pl.BlockSpec((bt, d), lambda i: (i, 0)),
            pl.BlockSpec((_E, d), lambda i: (0, 0)),
        ],
        out_specs=[
            pl.BlockSpec((bt, _TOPK), lambda i: (i, 0)),
            pl.BlockSpec((bt, _TOPK), lambda i: (i, 0)),
        ],
        out_shape=[
            jax.ShapeDtypeStruct((n, _TOPK), jnp.int32),
            jax.ShapeDtypeStruct((n, _TOPK), jnp.float32),
        ],
    )(flat, router_w)
    return idxs, ws


def _ffn_body(te_ref, act_ref, x_ref, gu_ref, dn_ref, w_ref, y_ref):
    del te_ref

    @pl.when(act_ref[pl.program_id(0)] > 0)
    def _():
        x = x_ref[...]
        gu_w = gu_ref[0]
        dn_w = dn_ref[0]
        gu = lax.dot_general(
            x, gu_w, (((1,), (1,)), ((), ())), preferred_element_type=jnp.float32
        )
        dff = gu.shape[1] // 2
        gate = gu[:, :dff]
        up = gu[:, dff:]
        h = gate * jax.nn.sigmoid(gate) * up
        y = lax.dot_general(
            h, dn_w, (((1,), (1,)), ((), ())), preferred_element_type=jnp.float32
        )
        y_ref[...] = y * w_ref[...]


def _grouped_ffn(x_sorted, gate_up_proj, down_proj, w_sorted, tile_expert, tile_active):
    cap, d = x_sorted.shape
    ntiles = cap // _TILE
    dff2 = gate_up_proj.shape[1]
    dff = down_proj.shape[2]
    grid_spec = pltpu.PrefetchScalarGridSpec(
        num_scalar_prefetch=2,
        grid=(ntiles,),
        in_specs=[
            pl.BlockSpec((_TILE, d), lambda t, te, act: (t, 0)),
            pl.BlockSpec((1, dff2, d), lambda t, te, act: (te[t], 0, 0)),
            pl.BlockSpec((1, d, dff), lambda t, te, act: (te[t], 0, 0)),
            pl.BlockSpec((_TILE, 1), lambda t, te, act: (t, 0)),
        ],
        out_specs=pl.BlockSpec((_TILE, d), lambda t, te, act: (t, 0)),
    )
    return pl.pallas_call(
        _ffn_body,
        grid_spec=grid_spec,
        out_shape=jax.ShapeDtypeStruct((cap, d), jnp.float32),
    )(tile_expert, tile_active, x_sorted, gate_up_proj, down_proj,
      w_sorted.reshape(cap, 1))


def kernel(hidden_states, gate_up_proj, down_proj, router_w):
    b, s, d = hidden_states.shape
    n = b * s
    npairs = n * _TOPK
    cap = npairs + _E * _TILE
    ntiles = cap // _TILE
    flat = hidden_states.reshape(n, d)

    idxs, ws = _route(flat, router_w)

    e_flat = idxs.reshape(-1)
    order = jnp.argsort(e_flat, stable=True)
    sorted_e = e_flat[order]
    counts = jnp.bincount(e_flat, length=_E)
    start = jnp.concatenate([jnp.zeros(1, jnp.int32), jnp.cumsum(counts)[:-1]])
    pad_counts = ((counts + _TILE - 1) // _TILE) * _TILE
    pad_cum = jnp.cumsum(pad_counts)
    pad_off = pad_cum - pad_counts
    total = pad_cum[-1]
    j = jnp.arange(npairs, dtype=jnp.int32)
    padpos = j + (pad_off - start)[sorted_e].astype(jnp.int32)
    src_token = jnp.zeros(cap, jnp.int32).at[padpos].set(
        (order // _TOPK).astype(jnp.int32))
    w_sorted = jnp.zeros(cap, jnp.float32).at[padpos].set(ws.reshape(-1)[order])
    pos_pair = jnp.zeros(npairs, jnp.int32).at[order].set(padpos)
    r0 = jnp.arange(ntiles, dtype=jnp.int32) * _TILE
    te_raw = jnp.searchsorted(pad_cum, r0, side="right").astype(jnp.int32)
    active = (r0 < total).astype(jnp.int32)
    last_tile = total // _TILE - 1
    e_last = te_raw[last_tile]
    tile_expert = jnp.where(active > 0, te_raw, e_last)

    x_sorted = flat[src_token]

    y = _grouped_ffn(x_sorted, gate_up_proj, down_proj, w_sorted,
                     tile_expert, active)

    pp = pos_pair.reshape(n, _TOPK)
    out = y[pp[:, 0]] + y[pp[:, 1]]
    return out.reshape(b, s, d)
